# baseline (speedup 1.0000x reference)
import functools

import jax
import jax.numpy as jnp
from jax import lax
from jax.experimental import pallas as pl
from jax.experimental.pallas import tpu as pltpu
from jax.experimental.pallas import tpu_sc as plsc

D = 32
B = 16384

_info = plsc.get_sparse_core_info()
_NC, _NS = _info.num_cores, _info.num_subcores
NW = _NC * _NS
B_PER_W = B // NW
CHUNK = 128
NCHUNK = B_PER_W // CHUNK

_mesh = plsc.VectorSubcoreMesh(core_axis_name="c", subcore_axis_name="s")


@functools.partial(
    pl.kernel,
    mesh=_mesh,
    out_type=jax.ShapeDtypeStruct((B, D), jnp.float32),
    scratch_types=[
        pltpu.VMEM((NCHUNK, CHUNK), jnp.int32),
        pltpu.VMEM((B_PER_W, D), jnp.float32),
        pltpu.SemaphoreType.DMA,
    ],
    compiler_params=pltpu.CompilerParams(use_tc_tiling_on_sc=False),
)
def _gather_kernel(idx_hbm, table_hbm, out_hbm, idx_v, rows_v, sem):
    wid = lax.axis_index("s") * _NC + lax.axis_index("c")
    base = wid * B_PER_W
    pltpu.sync_copy(idx_hbm.at[wid], idx_v)
    copies = [
        pltpu.async_copy(
            table_hbm.at[idx_v.at[j]],
            rows_v.at[pl.ds(j * CHUNK, CHUNK)],
            sem,
        )
        for j in range(NCHUNK)
    ]
    for c in copies:
        c.wait()
    pltpu.sync_copy(rows_v, out_hbm.at[pl.ds(base, B_PER_W)])


def kernel(node_idx, emb):
    idx3 = node_idx.astype(jnp.int32).reshape(NW, NCHUNK, CHUNK)
    return _gather_kernel(idx3, emb)


# speedup vs baseline: 3.5350x; 3.5350x over previous
import functools

import jax
import jax.numpy as jnp
from jax import lax
from jax.experimental import pallas as pl
from jax.experimental.pallas import tpu as pltpu
from jax.experimental.pallas import tpu_sc as plsc

D = 32
B = 16384
V = 1000000

_info = plsc.get_sparse_core_info()
_NC, _NS = _info.num_cores, _info.num_subcores
NW = _NC * _NS
BPW = B // NW
GS = 16
NBLK = BPW // 128

_mesh = plsc.VectorSubcoreMesh(core_axis_name="c", subcore_axis_name="s")


@functools.partial(
    pl.kernel,
    mesh=_mesh,
    out_type=jax.ShapeDtypeStruct((D, B), jnp.float32),
    scratch_types=[
        pltpu.VMEM((BPW,), jnp.int32),
        pltpu.VMEM((GS, D, 128), jnp.float32),
        pltpu.VMEM((D, 128), jnp.float32),
        pltpu.SemaphoreType.DMA,
        pltpu.SemaphoreType.DMA,
    ],
    compiler_params=pltpu.CompilerParams(needs_layout_passes=False),
)
def _gather_kernel(idx_hbm, embT_hbm, outT_hbm, idx_v, blk_v, ob_v, gsem, osem):
    wid = lax.axis_index("s") * _NC + lax.axis_index("c")
    base = wid * BPW
    pltpu.sync_copy(idx_hbm.at[pl.ds(base, BPW)], idx_v)
    iota = lax.iota(jnp.int32, 16)

    def block(blki, carry):
        bb = blki * 128
        for sub in range(128 // GS):
            rv = idx_v[pl.ds(bb + sub * GS, GS)]
            copies = []
            lanes = []
            for i in range(GS):
                r = rv[i]
                w0 = pl.multiple_of(
                    lax.shift_left(lax.shift_right_logical(r, 7), 7), 128
                )
                lanes.append(r - w0)
                copies.append(
                    pltpu.async_copy(
                        embT_hbm.at[:, pl.ds(w0, 128)], blk_v.at[i], gsem
                    )
                )
            for c in copies:
                c.wait()
            for i in range(GS):
                lane = jnp.full((16,), lanes[i], jnp.int32)
                row = jnp.full((16,), i, jnp.int32)
                col = jnp.full((16,), sub * GS + i, jnp.int32)
                lo = plsc.load_gather(blk_v, [row, iota, lane])
                hi = plsc.load_gather(blk_v, [row, iota + 16, lane])
                plsc.store_scatter(ob_v, [iota, col], lo)
                plsc.store_scatter(ob_v, [iota + 16, col], hi)
        pltpu.async_copy(
            ob_v, outT_hbm.at[:, pl.ds(base + bb, 128)], osem
        ).wait()
        return carry

    lax.fori_loop(0, NBLK, block, 0)


def kernel(node_idx, emb):
    outT = _gather_kernel(node_idx.astype(jnp.int32), emb.T)
    return outT.T
